# baseline (speedup 1.0000x reference)
import jax
import jax.numpy as jnp
from jax.experimental import pallas as pl
from jax.experimental.pallas import tpu as pltpu

_DIM = 2048
_HALF = _DIM // 2
_BS = 512


def _rope_kernel(labels_ref, x_ref, table_ref, o_ref, emb_ref):
    del labels_ref
    s_blk = pl.program_id(0)
    b = pl.program_id(1)

    @pl.when(b == 0)
    def _compute_sincos():
        pos = (_BS * s_blk).astype(jnp.float32) + jax.lax.broadcasted_iota(
            jnp.int32, (_BS, _HALF), 0
        ).astype(jnp.float32)
        d = jax.lax.broadcasted_iota(jnp.int32, (_BS, _HALF), 1).astype(jnp.float32)
        inv_freq = jnp.exp(d * (-jnp.log(10000.0) / _HALF))
        freqs = pos * inv_freq
        emb_ref[:, :_HALF] = jnp.sin(freqs)
        emb_ref[:, _HALF:] = jnp.cos(freqs)

    le = table_ref[0, 0, :]
    o_ref[0] = x_ref[0] + emb_ref[...] * le[None, :]


def kernel(x, labels, label_table):
    batch, seq, dim = x.shape
    assert dim == _DIM and seq % _BS == 0
    labels = labels.astype(jnp.int32)
    table3 = label_table.reshape(label_table.shape[0], 1, dim)
    grid = (seq // _BS, batch)
    return pl.pallas_call(
        _rope_kernel,
        grid_spec=pltpu.PrefetchScalarGridSpec(
            num_scalar_prefetch=1,
            grid=grid,
            in_specs=[
                pl.BlockSpec((1, _BS, _DIM), lambda s, b, labels: (b, s, 0)),
                pl.BlockSpec((1, 1, _DIM), lambda s, b, labels: (labels[b], 0, 0)),
            ],
            out_specs=pl.BlockSpec((1, _BS, _DIM), lambda s, b, labels: (b, s, 0)),
            scratch_shapes=[pltpu.VMEM((_BS, _DIM), jnp.float32)],
        ),
        out_shape=jax.ShapeDtypeStruct(x.shape, x.dtype),
        compiler_params=pltpu.CompilerParams(
            dimension_semantics=("parallel", "arbitrary"),
        ),
    )(labels, x, table3)


# speedup vs baseline: 1.5571x; 1.5571x over previous
import jax
import jax.numpy as jnp
from jax.experimental import pallas as pl
from jax.experimental.pallas import tpu as pltpu

_DIM = 2048
_HALF = _DIM // 2
_BS = 512


def _rope_kernel(labels_ref, x_ref, table_ref, o_ref, ksin_ref, kcos_ref, emb_ref):
    del labels_ref
    s_blk = pl.program_id(0)
    b = pl.program_id(1)

    @pl.when(jnp.logical_and(s_blk == 0, b == 0))
    def _fill_k_tables():
        k = jax.lax.broadcasted_iota(jnp.int32, (_BS, _HALF), 0).astype(jnp.float32)
        d = jax.lax.broadcasted_iota(jnp.int32, (_BS, _HALF), 1).astype(jnp.float32)
        inv_freq = jnp.exp(d * (-jnp.log(10000.0) / _HALF))
        ang = k * inv_freq
        ksin_ref[...] = jnp.sin(ang)
        kcos_ref[...] = jnp.cos(ang)

    @pl.when(b == 0)
    def _compute_block_sincos():
        d = jax.lax.broadcasted_iota(jnp.int32, (1, _HALF), 1).astype(jnp.float32)
        inv_freq = jnp.exp(d * (-jnp.log(10000.0) / _HALF))
        ang0 = (_BS * s_blk).astype(jnp.float32) * inv_freq
        sin0 = jnp.sin(ang0)
        cos0 = jnp.cos(ang0)
        ksin = ksin_ref[...]
        kcos = kcos_ref[...]
        emb_ref[:, :_HALF] = ksin * cos0 + kcos * sin0
        emb_ref[:, _HALF:] = kcos * cos0 - ksin * sin0

    le = table_ref[0, 0, :]
    o_ref[0] = x_ref[0] + emb_ref[...] * le[None, :]


def kernel(x, labels, label_table):
    batch, seq, dim = x.shape
    assert dim == _DIM and seq % _BS == 0
    labels = labels.astype(jnp.int32)
    table3 = label_table.reshape(label_table.shape[0], 1, dim)
    grid = (seq // _BS, batch)
    return pl.pallas_call(
        _rope_kernel,
        grid_spec=pltpu.PrefetchScalarGridSpec(
            num_scalar_prefetch=1,
            grid=grid,
            in_specs=[
                pl.BlockSpec((1, _BS, _DIM), lambda s, b, labels: (b, s, 0)),
                pl.BlockSpec((1, 1, _DIM), lambda s, b, labels: (labels[b], 0, 0)),
            ],
            out_specs=pl.BlockSpec((1, _BS, _DIM), lambda s, b, labels: (b, s, 0)),
            scratch_shapes=[
                pltpu.VMEM((_BS, _HALF), jnp.float32),
                pltpu.VMEM((_BS, _HALF), jnp.float32),
                pltpu.VMEM((_BS, _DIM), jnp.float32),
            ],
        ),
        out_shape=jax.ShapeDtypeStruct(x.shape, x.dtype),
        compiler_params=pltpu.CompilerParams(
            dimension_semantics=("arbitrary", "arbitrary"),
        ),
    )(labels, x, table3)
